# Initial kernel scaffold; baseline (speedup 1.0000x reference)
import jax
import jax.numpy as jnp
from jax.experimental import pallas as pl


def kernel(nodes, feat, src_idx, pos_idx, neg_idx, neighbor_0, neighbor_1, neighbor_2, W_self_0, W_neigh_0, W_self_1, W_neigh_1):
    raise NotImplementedError("write your pallas kernel here")


# speedup vs baseline: 2.3367x; 2.3367x over previous
import functools

import jax
import jax.numpy as jnp
from jax import lax
from jax.experimental import pallas as pl
from jax.experimental.pallas import tpu as pltpu
from jax.experimental.pallas import tpu_sc as plsc

N_FEAT = 100000
IN_DIM = 256
N0 = 1024
S1 = 25
S2 = 10
D1 = 128
D2 = 128
N_PAIR = 512
N_NEG = 512

NW = 32
G2_PER_W = (N0 * S1) // NW
CH_G = 16
CH_R = CH_G * S2
N_CH2 = G2_PER_W // CH_G
H1_PER_W = (N0 * S1) // NW
N_CH1 = H1_PER_W // CH_R
H0_PER_W = N0 // NW


def _sc_body(feat, n0, n1, n2, h0_out, h1_out, s2_out, idx_a, idx_b, rows, osum, sem):
    c = lax.axis_index("c")
    s = lax.axis_index("s")
    wid = s * 2 + c

    def gather_chunk(src_idx_hbm, base):
        pltpu.sync_copy(src_idx_hbm.at[pl.ds(base, 128)], idx_a)
        pltpu.sync_copy(src_idx_hbm.at[pl.ds(base + 128, 32)], idx_b)
        cp1 = pltpu.async_copy(feat.at[idx_a], rows.at[pl.ds(0, 128)], sem)
        cp2 = pltpu.async_copy(feat.at[idx_b], rows.at[pl.ds(128, 32)], sem)
        cp1.wait()
        cp2.wait()

    def h2_iter(i, carry):
        base = wid * (G2_PER_W * S2) + i * CH_R
        gather_chunk(n2, base)

        def grp(g, carry2):
            r0 = g * S2
            for ch in range(IN_DIM // 16):
                sl = pl.ds(ch * 16, 16)
                acc = rows[r0, sl]
                for k in range(1, S2):
                    acc = acc + rows[r0 + k, sl]
                osum[g, sl] = acc
            return carry2

        lax.fori_loop(0, CH_G, grp, 0)
        pltpu.sync_copy(osum, s2_out.at[pl.ds(wid * G2_PER_W + i * CH_G, CH_G)])
        return carry

    lax.fori_loop(0, N_CH2, h2_iter, 0)

    def h1_iter(i, carry):
        base = wid * H1_PER_W + i * CH_R
        gather_chunk(n1, base)
        pltpu.sync_copy(rows, h1_out.at[pl.ds(base, CH_R)])
        return carry

    lax.fori_loop(0, N_CH1, h1_iter, 0)

    pltpu.sync_copy(n0.at[pl.ds(wid * H0_PER_W, H0_PER_W)], idx_b)
    pltpu.async_copy(feat.at[idx_b], rows.at[pl.ds(0, H0_PER_W)], sem).wait()
    pltpu.sync_copy(rows.at[pl.ds(0, H0_PER_W)],
                    h0_out.at[pl.ds(wid * H0_PER_W, H0_PER_W)])


_sc_gather = functools.partial(
    pl.kernel,
    mesh=plsc.VectorSubcoreMesh(core_axis_name="c", subcore_axis_name="s"),
    out_type=(
        jax.ShapeDtypeStruct((N0, IN_DIM), jnp.float32),
        jax.ShapeDtypeStruct((N0 * S1, IN_DIM), jnp.float32),
        jax.ShapeDtypeStruct((N0 * S1, IN_DIM), jnp.float32),
    ),
    scratch_types=[
        pltpu.VMEM((128,), jnp.int32),
        pltpu.VMEM((32,), jnp.int32),
        pltpu.VMEM((CH_R, IN_DIM), jnp.float32),
        pltpu.VMEM((CH_G, IN_DIM), jnp.float32),
        pltpu.SemaphoreType.DMA,
    ],
)(_sc_body)


BG = 128
BR = BG * S1


def _tc1_body(h1_ref, s2_ref, ws0_ref, wn0_ref, m1_ref, n1s_ref):
    h1b = h1_ref[...]
    xs = jnp.dot(h1b, ws0_ref[...], preferred_element_type=jnp.float32)
    xn = jnp.dot(s2_ref[...] * (1.0 / S2), wn0_ref[...],
                 preferred_element_type=jnp.float32)
    x = jnp.maximum(jnp.concatenate([xs, xn], axis=1), 0.0)
    m1_ref[...] = jnp.sum(x.reshape(BG, S1, 2 * D1), axis=1) * (1.0 / S1)
    n1s_ref[...] = jnp.sum(h1b.reshape(BG, S1, IN_DIM), axis=1)


def _tc1(h1, s2, Ws0, Wn0):
    return pl.pallas_call(
        _tc1_body,
        grid=(N0 // BG,),
        in_specs=[
            pl.BlockSpec((BR, IN_DIM), lambda i: (i, 0)),
            pl.BlockSpec((BR, IN_DIM), lambda i: (i, 0)),
            pl.BlockSpec((IN_DIM, D1), lambda i: (0, 0)),
            pl.BlockSpec((IN_DIM, D1), lambda i: (0, 0)),
        ],
        out_specs=(
            pl.BlockSpec((BG, 2 * D1), lambda i: (i, 0)),
            pl.BlockSpec((BG, IN_DIM), lambda i: (i, 0)),
        ),
        out_shape=(
            jax.ShapeDtypeStruct((N0, 2 * D1), jnp.float32),
            jax.ShapeDtypeStruct((N0, IN_DIM), jnp.float32),
        ),
    )(h1, s2, Ws0, Wn0)


def _softplus(x):
    return jnp.maximum(x, 0.0) + jnp.log1p(jnp.exp(-jnp.abs(x)))


def _tc2_body(h0_ref, n1s_ref, m1_ref, ws0_ref, wn0_ref, ws1_ref, wn1_ref,
              si_ref, pi_ref, ni_ref, src_ref, loss_ref, mrr_ref):
    xs = jnp.dot(h0_ref[...], ws0_ref[...], preferred_element_type=jnp.float32)
    xn = jnp.dot(n1s_ref[...] * (1.0 / S1), wn0_ref[...],
                 preferred_element_type=jnp.float32)
    l1h0 = jnp.maximum(jnp.concatenate([xs, xn], axis=1), 0.0)
    ys = jnp.dot(l1h0, ws1_ref[...], preferred_element_type=jnp.float32)
    yn = jnp.dot(m1_ref[...], wn1_ref[...], preferred_element_type=jnp.float32)
    out = jnp.concatenate([ys, yn], axis=1)
    sq = jnp.maximum(jnp.sum(out * out, axis=1, keepdims=True), 1e-12)
    out = out * lax.rsqrt(sq)
    cols = lax.broadcasted_iota(jnp.int32, (N_PAIR, N0), 1)
    src_emb = jnp.dot((cols == si_ref[...]).astype(jnp.float32), out,
                      preferred_element_type=jnp.float32,
                      precision=lax.Precision.HIGHEST)
    pos_emb = jnp.dot((cols == pi_ref[...]).astype(jnp.float32), out,
                      preferred_element_type=jnp.float32,
                      precision=lax.Precision.HIGHEST)
    neg_emb = jnp.dot((cols == ni_ref[...]).astype(jnp.float32), out,
                      preferred_element_type=jnp.float32,
                      precision=lax.Precision.HIGHEST)
    src_ref[...] = src_emb
    aff = jnp.sum(src_emb * pos_emb, axis=1, keepdims=True)
    neg_aff = lax.dot_general(src_emb, neg_emb, (((1,), (1,)), ((), ())),
                              preferred_element_type=jnp.float32)
    rank = jnp.sum((neg_aff >= aff).astype(jnp.int32), axis=1, keepdims=True)
    mrr_ref[...] = jnp.mean(1.0 / (rank.astype(jnp.float32) + 1.0)).reshape(1, 1)
    loss = jnp.sum(_softplus(-aff)) + jnp.sum(_softplus(neg_aff))
    loss_ref[...] = (loss * (1.0 / N_PAIR)).reshape(1, 1)


def _tc2(h0, n1s, m1, Ws0, Wn0, Ws1, Wn1, si, pi, ni):
    return pl.pallas_call(
        _tc2_body,
        out_shape=(
            jax.ShapeDtypeStruct((N_PAIR, D2 * 2), jnp.float32),
            jax.ShapeDtypeStruct((1, 1), jnp.float32),
            jax.ShapeDtypeStruct((1, 1), jnp.float32),
        ),
    )(h0, n1s, m1, Ws0, Wn0, Ws1, Wn1, si, pi, ni)


def kernel(nodes, feat, src_idx, pos_idx, neg_idx, neighbor_0, neighbor_1,
           neighbor_2, W_self_0, W_neigh_0, W_self_1, W_neigh_1):
    h0, h1, s2 = _sc_gather(feat, neighbor_0, neighbor_1, neighbor_2)
    m1, n1s = _tc1(h1, s2, W_self_0, W_neigh_0)
    src_emb, loss, mrr = _tc2(
        h0, n1s, m1, W_self_0, W_neigh_0, W_self_1, W_neigh_1,
        src_idx.reshape(N_PAIR, 1), pos_idx.reshape(N_PAIR, 1),
        neg_idx.reshape(N_NEG, 1))
    return src_emb, loss[0, 0], mrr[0, 0]


# speedup vs baseline: 3.5725x; 1.5288x over previous
import functools

import jax
import jax.numpy as jnp
from jax import lax
from jax.experimental import pallas as pl
from jax.experimental.pallas import tpu as pltpu
from jax.experimental.pallas import tpu_sc as plsc

N_FEAT = 100000
IN_DIM = 256
N0 = 1024
S1 = 25
S2 = 10
D1 = 128
D2 = 128
N_PAIR = 512
N_NEG = 512

NW = 32
G2_PER_W = (N0 * S1) // NW
CH_G = 16
CH_R = CH_G * S2
N_CH2 = G2_PER_W // CH_G
H1_PER_W = (N0 * S1) // NW
N_CH1 = H1_PER_W // CH_R
H0_PER_W = N0 // NW


IDX2_PER_W = G2_PER_W * S2


def _sc_body(feat, n0, n1, n2, h0_out, h1_out, s2_out,
             idxbuf, rows_a, rows_b, osum_a, osum_b,
             sem_a, sem_b, sem_oa, sem_ob):
    c = lax.axis_index("c")
    s = lax.axis_index("s")
    wid = s * 2 + c

    def issue(rows, sem, off):
        pltpu.async_copy(feat.at[idxbuf.at[pl.ds(off, 128)]],
                         rows.at[pl.ds(0, 128)], sem)
        pltpu.async_copy(feat.at[idxbuf.at[pl.ds(off + 128, 32)]],
                         rows.at[pl.ds(128, 32)], sem)

    def wait_gather(rows, sem, off):
        pltpu.make_async_copy(feat.at[idxbuf.at[pl.ds(off, 128)]],
                              rows.at[pl.ds(0, 128)], sem).wait()
        pltpu.make_async_copy(feat.at[idxbuf.at[pl.ds(off + 128, 32)]],
                              rows.at[pl.ds(128, 32)], sem).wait()

    def reduce_into(rows, osum):
        def grp(g, carry):
            r0 = g * S2
            for ch in range(IN_DIM // 16):
                sl = pl.ds(ch * 16, 16)
                acc = rows[r0, sl]
                for k in range(1, S2):
                    acc = acc + rows[r0 + k, sl]
                osum[g, sl] = acc
            return carry

        lax.fori_loop(0, CH_G, grp, 0)

    def s2_dst(i):
        return s2_out.at[pl.ds(wid * G2_PER_W + i * CH_G, CH_G)]

    pltpu.sync_copy(n2.at[pl.ds(wid * IDX2_PER_W, IDX2_PER_W)], idxbuf)
    issue(rows_a, sem_a, 0)
    NJ = N_CH2 // 2

    def pipe(j, carry):
        i0 = 2 * j
        issue(rows_b, sem_b, (i0 + 1) * CH_R)
        wait_gather(rows_a, sem_a, i0 * CH_R)

        @pl.when(j > 0)
        def _():
            pltpu.make_async_copy(osum_a, s2_dst(i0 - 2), sem_oa).wait()

        reduce_into(rows_a, osum_a)
        pltpu.async_copy(osum_a, s2_dst(i0), sem_oa)

        @pl.when(j < NJ - 1)
        def _():
            issue(rows_a, sem_a, (i0 + 2) * CH_R)

        wait_gather(rows_b, sem_b, (i0 + 1) * CH_R)

        @pl.when(j > 0)
        def _():
            pltpu.make_async_copy(osum_b, s2_dst(i0 - 1), sem_ob).wait()

        reduce_into(rows_b, osum_b)
        pltpu.async_copy(osum_b, s2_dst(i0 + 1), sem_ob)
        return carry

    lax.fori_loop(0, NJ, pipe, 0)
    pltpu.make_async_copy(osum_a, s2_dst(N_CH2 - 2), sem_oa).wait()
    pltpu.make_async_copy(osum_b, s2_dst(N_CH2 - 1), sem_ob).wait()

    pltpu.sync_copy(n1.at[pl.ds(wid * H1_PER_W, H1_PER_W)],
                    idxbuf.at[pl.ds(0, H1_PER_W)])
    prev_out = [None, None]
    for ci in range(N_CH1):
        rows, sem, osem = ((rows_a, sem_a, sem_oa) if ci % 2 == 0
                           else (rows_b, sem_b, sem_ob))
        off = ci * CH_R
        if prev_out[ci % 2] is not None:
            prev_out[ci % 2].wait()
        cp1 = pltpu.async_copy(feat.at[idxbuf.at[pl.ds(off, 128)]],
                               rows.at[pl.ds(0, 128)], sem)
        cp2 = pltpu.async_copy(feat.at[idxbuf.at[pl.ds(off + 128, 32)]],
                               rows.at[pl.ds(128, 32)], sem)
        cp1.wait()
        cp2.wait()
        prev_out[ci % 2] = pltpu.async_copy(
            rows, h1_out.at[pl.ds(wid * H1_PER_W + off, CH_R)], osem)
    for h in prev_out:
        h.wait()

    pltpu.sync_copy(n0.at[pl.ds(wid * H0_PER_W, H0_PER_W)],
                    idxbuf.at[pl.ds(0, H0_PER_W)])
    pltpu.async_copy(feat.at[idxbuf.at[pl.ds(0, H0_PER_W)]],
                     rows_a.at[pl.ds(0, H0_PER_W)], sem_a).wait()
    pltpu.sync_copy(rows_a.at[pl.ds(0, H0_PER_W)],
                    h0_out.at[pl.ds(wid * H0_PER_W, H0_PER_W)])


_sc_gather = functools.partial(
    pl.kernel,
    mesh=plsc.VectorSubcoreMesh(core_axis_name="c", subcore_axis_name="s"),
    out_type=(
        jax.ShapeDtypeStruct((N0, IN_DIM), jnp.float32),
        jax.ShapeDtypeStruct((N0 * S1, IN_DIM), jnp.float32),
        jax.ShapeDtypeStruct((N0 * S1, IN_DIM), jnp.float32),
    ),
    scratch_types=[
        pltpu.VMEM((IDX2_PER_W,), jnp.int32),
        pltpu.VMEM((CH_R, IN_DIM), jnp.float32),
        pltpu.VMEM((CH_R, IN_DIM), jnp.float32),
        pltpu.VMEM((CH_G, IN_DIM), jnp.float32),
        pltpu.VMEM((CH_G, IN_DIM), jnp.float32),
        pltpu.SemaphoreType.DMA,
        pltpu.SemaphoreType.DMA,
        pltpu.SemaphoreType.DMA,
        pltpu.SemaphoreType.DMA,
    ],
)(_sc_body)


BG = 128
BR = BG * S1


def _tc1_body(h1_ref, s2_ref, ws0_ref, wn0_ref, m1_ref, n1s_ref):
    h1b = h1_ref[...]
    xs = jnp.dot(h1b, ws0_ref[...], preferred_element_type=jnp.float32)
    xn = jnp.dot(s2_ref[...] * (1.0 / S2), wn0_ref[...],
                 preferred_element_type=jnp.float32)
    x = jnp.maximum(jnp.concatenate([xs, xn], axis=1), 0.0)
    m1_ref[...] = jnp.sum(x.reshape(BG, S1, 2 * D1), axis=1) * (1.0 / S1)
    n1s_ref[...] = jnp.sum(h1b.reshape(BG, S1, IN_DIM), axis=1)


def _tc1(h1, s2, Ws0, Wn0):
    return pl.pallas_call(
        _tc1_body,
        grid=(N0 // BG,),
        in_specs=[
            pl.BlockSpec((BR, IN_DIM), lambda i: (i, 0)),
            pl.BlockSpec((BR, IN_DIM), lambda i: (i, 0)),
            pl.BlockSpec((IN_DIM, D1), lambda i: (0, 0)),
            pl.BlockSpec((IN_DIM, D1), lambda i: (0, 0)),
        ],
        out_specs=(
            pl.BlockSpec((BG, 2 * D1), lambda i: (i, 0)),
            pl.BlockSpec((BG, IN_DIM), lambda i: (i, 0)),
        ),
        out_shape=(
            jax.ShapeDtypeStruct((N0, 2 * D1), jnp.float32),
            jax.ShapeDtypeStruct((N0, IN_DIM), jnp.float32),
        ),
    )(h1, s2, Ws0, Wn0)


def _softplus(x):
    return jnp.maximum(x, 0.0) + jnp.log1p(jnp.exp(-jnp.abs(x)))


def _tc2_body(h0_ref, n1s_ref, m1_ref, ws0_ref, wn0_ref, ws1_ref, wn1_ref,
              si_ref, pi_ref, ni_ref, src_ref, loss_ref, mrr_ref):
    xs = jnp.dot(h0_ref[...], ws0_ref[...], preferred_element_type=jnp.float32)
    xn = jnp.dot(n1s_ref[...] * (1.0 / S1), wn0_ref[...],
                 preferred_element_type=jnp.float32)
    l1h0 = jnp.maximum(jnp.concatenate([xs, xn], axis=1), 0.0)
    ys = jnp.dot(l1h0, ws1_ref[...], preferred_element_type=jnp.float32)
    yn = jnp.dot(m1_ref[...], wn1_ref[...], preferred_element_type=jnp.float32)
    out = jnp.concatenate([ys, yn], axis=1)
    sq = jnp.maximum(jnp.sum(out * out, axis=1, keepdims=True), 1e-12)
    out = out * lax.rsqrt(sq)
    cols = lax.broadcasted_iota(jnp.int32, (N_PAIR, N0), 1)
    src_emb = jnp.dot((cols == si_ref[...]).astype(jnp.float32), out,
                      preferred_element_type=jnp.float32,
                      precision=lax.Precision.HIGHEST)
    pos_emb = jnp.dot((cols == pi_ref[...]).astype(jnp.float32), out,
                      preferred_element_type=jnp.float32,
                      precision=lax.Precision.HIGHEST)
    neg_emb = jnp.dot((cols == ni_ref[...]).astype(jnp.float32), out,
                      preferred_element_type=jnp.float32,
                      precision=lax.Precision.HIGHEST)
    src_ref[...] = src_emb
    aff = jnp.sum(src_emb * pos_emb, axis=1, keepdims=True)
    neg_aff = lax.dot_general(src_emb, neg_emb, (((1,), (1,)), ((), ())),
                              preferred_element_type=jnp.float32)
    rank = jnp.sum((neg_aff >= aff).astype(jnp.int32), axis=1, keepdims=True)
    mrr_ref[...] = jnp.mean(1.0 / (rank.astype(jnp.float32) + 1.0)).reshape(1, 1)
    loss = jnp.sum(_softplus(-aff)) + jnp.sum(_softplus(neg_aff))
    loss_ref[...] = (loss * (1.0 / N_PAIR)).reshape(1, 1)


def _tc2(h0, n1s, m1, Ws0, Wn0, Ws1, Wn1, si, pi, ni):
    return pl.pallas_call(
        _tc2_body,
        out_shape=(
            jax.ShapeDtypeStruct((N_PAIR, D2 * 2), jnp.float32),
            jax.ShapeDtypeStruct((1, 1), jnp.float32),
            jax.ShapeDtypeStruct((1, 1), jnp.float32),
        ),
    )(h0, n1s, m1, Ws0, Wn0, Ws1, Wn1, si, pi, ni)


def kernel(nodes, feat, src_idx, pos_idx, neg_idx, neighbor_0, neighbor_1,
           neighbor_2, W_self_0, W_neigh_0, W_self_1, W_neigh_1):
    h0, h1, s2 = _sc_gather(feat, neighbor_0, neighbor_1, neighbor_2)
    m1, n1s = _tc1(h1, s2, W_self_0, W_neigh_0)
    src_emb, loss, mrr = _tc2(
        h0, n1s, m1, W_self_0, W_neigh_0, W_self_1, W_neigh_1,
        src_idx.reshape(N_PAIR, 1), pos_idx.reshape(N_PAIR, 1),
        neg_idx.reshape(N_NEG, 1))
    return src_emb, loss[0, 0], mrr[0, 0]


# speedup vs baseline: 4.8341x; 1.3532x over previous
import functools

import jax
import jax.numpy as jnp
from jax import lax
from jax.experimental import pallas as pl
from jax.experimental.pallas import tpu as pltpu
from jax.experimental.pallas import tpu_sc as plsc

N_FEAT = 100000
IN_DIM = 256
N0 = 1024
S1 = 25
S2 = 10
D1 = 128
D2 = 128
N_PAIR = 512
N_NEG = 512

NW = 32
G2_PER_W = (N0 * S1) // NW
CH_G = 16
CH_R = CH_G * S2
N_CH2 = G2_PER_W // CH_G
H1_PER_W = (N0 * S1) // NW
CH1_R = 80
N_CH1 = H1_PER_W // CH1_R
H0_PER_W = N0 // NW
IDX2_PER_W = G2_PER_W * S2


FB = N_FEAT // 10


def _tc0_body(feat_ref, wn0_ref, g_ref):
    g_ref[...] = jnp.dot(feat_ref[...], wn0_ref[...],
                         preferred_element_type=jnp.float32)


def _tc0(feat, Wn0):
    return pl.pallas_call(
        _tc0_body,
        grid=(N_FEAT // FB,),
        in_specs=[
            pl.BlockSpec((FB, IN_DIM), lambda i: (i, 0)),
            pl.BlockSpec((IN_DIM, D1), lambda i: (0, 0)),
        ],
        out_specs=pl.BlockSpec((FB, D1), lambda i: (i, 0)),
        out_shape=jax.ShapeDtypeStruct((N_FEAT, D1), jnp.float32),
    )(feat, Wn0)


def _sc_body(feat, gtab, n0, n1, n2, h0_out, h1_out, s2_out,
             idxbuf, r2a, r2b, r1a, r1b, osum_a, osum_b,
             sem_a, sem_b, sem_oa, sem_ob):
    c = lax.axis_index("c")
    s = lax.axis_index("s")
    wid = s * 2 + c

    def issue(rows, sem, off):
        pltpu.async_copy(gtab.at[idxbuf.at[pl.ds(off, 128)]],
                         rows.at[pl.ds(0, 128)], sem)
        pltpu.async_copy(gtab.at[idxbuf.at[pl.ds(off + 128, 32)]],
                         rows.at[pl.ds(128, 32)], sem)

    def wait_gather(rows, sem, off):
        pltpu.make_async_copy(gtab.at[idxbuf.at[pl.ds(off, 128)]],
                              rows.at[pl.ds(0, 128)], sem).wait()
        pltpu.make_async_copy(gtab.at[idxbuf.at[pl.ds(off + 128, 32)]],
                              rows.at[pl.ds(128, 32)], sem).wait()

    def reduce_into(rows, osum):
        def grp(g, carry):
            r0 = g * S2
            for ch in range(D1 // 16):
                sl = pl.ds(ch * 16, 16)
                acc = rows[r0, sl]
                for k in range(1, S2):
                    acc = acc + rows[r0 + k, sl]
                osum[g, sl] = acc
            return carry

        lax.fori_loop(0, CH_G, grp, 0)

    def s2_dst(i):
        return s2_out.at[pl.ds(wid * G2_PER_W + i * CH_G, CH_G)]

    pltpu.sync_copy(n2.at[pl.ds(wid * IDX2_PER_W, IDX2_PER_W)], idxbuf)
    issue(r2a, sem_a, 0)
    NJ = N_CH2 // 2

    def pipe(j, carry):
        i0 = 2 * j
        issue(r2b, sem_b, (i0 + 1) * CH_R)
        wait_gather(r2a, sem_a, i0 * CH_R)

        @pl.when(j > 0)
        def _():
            pltpu.make_async_copy(osum_a, s2_dst(i0 - 2), sem_oa).wait()

        reduce_into(r2a, osum_a)
        pltpu.async_copy(osum_a, s2_dst(i0), sem_oa)

        @pl.when(j < NJ - 1)
        def _():
            issue(r2a, sem_a, (i0 + 2) * CH_R)

        wait_gather(r2b, sem_b, (i0 + 1) * CH_R)

        @pl.when(j > 0)
        def _():
            pltpu.make_async_copy(osum_b, s2_dst(i0 - 1), sem_ob).wait()

        reduce_into(r2b, osum_b)
        pltpu.async_copy(osum_b, s2_dst(i0 + 1), sem_ob)
        return carry

    lax.fori_loop(0, NJ, pipe, 0)
    pltpu.make_async_copy(osum_a, s2_dst(N_CH2 - 2), sem_oa).wait()
    pltpu.make_async_copy(osum_b, s2_dst(N_CH2 - 1), sem_ob).wait()

    pltpu.sync_copy(n1.at[pl.ds(wid * H1_PER_W, H1_PER_W)],
                    idxbuf.at[pl.ds(0, H1_PER_W)])
    prev_out = [None, None]
    for ci in range(N_CH1):
        rows, sem, osem = ((r1a, sem_a, sem_oa) if ci % 2 == 0
                           else (r1b, sem_b, sem_ob))
        off = ci * CH1_R
        if prev_out[ci % 2] is not None:
            prev_out[ci % 2].wait()
        cp = pltpu.async_copy(feat.at[idxbuf.at[pl.ds(off, CH1_R)]], rows, sem)
        cp.wait()
        prev_out[ci % 2] = pltpu.async_copy(
            rows, h1_out.at[pl.ds(wid * H1_PER_W + off, CH1_R)], osem)
    for h in prev_out:
        h.wait()

    pltpu.sync_copy(n0.at[pl.ds(wid * H0_PER_W, H0_PER_W)],
                    idxbuf.at[pl.ds(0, H0_PER_W)])
    pltpu.async_copy(feat.at[idxbuf.at[pl.ds(0, H0_PER_W)]],
                     r1a.at[pl.ds(0, H0_PER_W)], sem_a).wait()
    pltpu.sync_copy(r1a.at[pl.ds(0, H0_PER_W)],
                    h0_out.at[pl.ds(wid * H0_PER_W, H0_PER_W)])


_sc_gather = functools.partial(
    pl.kernel,
    mesh=plsc.VectorSubcoreMesh(core_axis_name="c", subcore_axis_name="s"),
    out_type=(
        jax.ShapeDtypeStruct((N0, IN_DIM), jnp.float32),
        jax.ShapeDtypeStruct((N0 * S1, IN_DIM), jnp.float32),
        jax.ShapeDtypeStruct((N0 * S1, D1), jnp.float32),
    ),
    scratch_types=[
        pltpu.VMEM((IDX2_PER_W,), jnp.int32),
        pltpu.VMEM((CH_R, D1), jnp.float32),
        pltpu.VMEM((CH_R, D1), jnp.float32),
        pltpu.VMEM((CH1_R, IN_DIM), jnp.float32),
        pltpu.VMEM((CH1_R, IN_DIM), jnp.float32),
        pltpu.VMEM((CH_G, D1), jnp.float32),
        pltpu.VMEM((CH_G, D1), jnp.float32),
        pltpu.SemaphoreType.DMA,
        pltpu.SemaphoreType.DMA,
        pltpu.SemaphoreType.DMA,
        pltpu.SemaphoreType.DMA,
    ],
)(_sc_body)


BG = 128
BR = BG * S1


def _tc1_body(h1_ref, s2_ref, ws0_ref, m1_ref, n1s_ref):
    h1b = h1_ref[...]
    xs = jnp.dot(h1b, ws0_ref[...], preferred_element_type=jnp.float32)
    x = jnp.maximum(jnp.concatenate([xs, s2_ref[...] * (1.0 / S2)], axis=1), 0.0)
    col_grp = lax.broadcasted_iota(jnp.int32, (BG, BR), 1) // S1
    row_grp = lax.broadcasted_iota(jnp.int32, (BG, BR), 0)
    a = (col_grp == row_grp).astype(jnp.float32)
    m1_ref[...] = jnp.dot(a, x, preferred_element_type=jnp.float32) * (1.0 / S1)
    n1s_ref[...] = jnp.dot(a, h1b, preferred_element_type=jnp.float32)


def _tc1(h1, s2, Ws0):
    return pl.pallas_call(
        _tc1_body,
        grid=(N0 // BG,),
        in_specs=[
            pl.BlockSpec((BR, IN_DIM), lambda i: (i, 0)),
            pl.BlockSpec((BR, D1), lambda i: (i, 0)),
            pl.BlockSpec((IN_DIM, D1), lambda i: (0, 0)),
        ],
        out_specs=(
            pl.BlockSpec((BG, 2 * D1), lambda i: (i, 0)),
            pl.BlockSpec((BG, IN_DIM), lambda i: (i, 0)),
        ),
        out_shape=(
            jax.ShapeDtypeStruct((N0, 2 * D1), jnp.float32),
            jax.ShapeDtypeStruct((N0, IN_DIM), jnp.float32),
        ),
    )(h1, s2, Ws0)


def _softplus(x):
    return jnp.maximum(x, 0.0) + jnp.log1p(jnp.exp(-jnp.abs(x)))


def _tc2_body(h0_ref, n1s_ref, m1_ref, ws0_ref, wn0_ref, ws1_ref, wn1_ref,
              si_ref, pi_ref, ni_ref, src_ref, loss_ref, mrr_ref):
    xs = jnp.dot(h0_ref[...], ws0_ref[...], preferred_element_type=jnp.float32)
    xn = jnp.dot(n1s_ref[...] * (1.0 / S1), wn0_ref[...],
                 preferred_element_type=jnp.float32)
    l1h0 = jnp.maximum(jnp.concatenate([xs, xn], axis=1), 0.0)
    ys = jnp.dot(l1h0, ws1_ref[...], preferred_element_type=jnp.float32)
    yn = jnp.dot(m1_ref[...], wn1_ref[...], preferred_element_type=jnp.float32)
    out = jnp.concatenate([ys, yn], axis=1)
    sq = jnp.maximum(jnp.sum(out * out, axis=1, keepdims=True), 1e-12)
    out = out * lax.rsqrt(sq)
    cols = lax.broadcasted_iota(jnp.int32, (N_PAIR, N0), 1)
    src_emb = jnp.dot((cols == si_ref[...]).astype(jnp.float32), out,
                      preferred_element_type=jnp.float32,
                      precision=lax.Precision.HIGHEST)
    pos_emb = jnp.dot((cols == pi_ref[...]).astype(jnp.float32), out,
                      preferred_element_type=jnp.float32,
                      precision=lax.Precision.HIGHEST)
    neg_emb = jnp.dot((cols == ni_ref[...]).astype(jnp.float32), out,
                      preferred_element_type=jnp.float32,
                      precision=lax.Precision.HIGHEST)
    src_ref[...] = src_emb
    aff = jnp.sum(src_emb * pos_emb, axis=1, keepdims=True)
    neg_aff = lax.dot_general(src_emb, neg_emb, (((1,), (1,)), ((), ())),
                              preferred_element_type=jnp.float32)
    rank = jnp.sum((neg_aff >= aff).astype(jnp.int32), axis=1, keepdims=True)
    mrr_ref[...] = jnp.mean(1.0 / (rank.astype(jnp.float32) + 1.0)).reshape(1, 1)
    loss = jnp.sum(_softplus(-aff)) + jnp.sum(_softplus(neg_aff))
    loss_ref[...] = (loss * (1.0 / N_PAIR)).reshape(1, 1)


def _tc2(h0, n1s, m1, Ws0, Wn0, Ws1, Wn1, si, pi, ni):
    return pl.pallas_call(
        _tc2_body,
        out_shape=(
            jax.ShapeDtypeStruct((N_PAIR, D2 * 2), jnp.float32),
            jax.ShapeDtypeStruct((1, 1), jnp.float32),
            jax.ShapeDtypeStruct((1, 1), jnp.float32),
        ),
    )(h0, n1s, m1, Ws0, Wn0, Ws1, Wn1, si, pi, ni)


def kernel(nodes, feat, src_idx, pos_idx, neg_idx, neighbor_0, neighbor_1,
           neighbor_2, W_self_0, W_neigh_0, W_self_1, W_neigh_1):
    gtab = _tc0(feat, W_neigh_0)
    h0, h1, s2g = _sc_gather(feat, gtab, neighbor_0, neighbor_1, neighbor_2)
    m1, n1s = _tc1(h1, s2g, W_self_0)
    src_emb, loss, mrr = _tc2(
        h0, n1s, m1, W_self_0, W_neigh_0, W_self_1, W_neigh_1,
        src_idx.reshape(N_PAIR, 1), pos_idx.reshape(N_PAIR, 1),
        neg_idx.reshape(N_NEG, 1))
    return src_emb, loss[0, 0], mrr[0, 0]
